# Initial kernel scaffold; baseline (speedup 1.0000x reference)
import jax
import jax.numpy as jnp
from jax.experimental import pallas as pl


def kernel(emb0_weight, emb1_weight, data):
    raise NotImplementedError("write your pallas kernel here")


# speedup vs baseline: 1.5334x; 1.5334x over previous
import functools

import jax
import jax.numpy as jnp
from jax import lax
from jax.experimental import pallas as pl
from jax.experimental.pallas import tpu as pltpu
from jax.experimental.pallas import tpu_sc as plsc

V = 1000000
D = 64
W = 5
NEG = 5
B = 16384

_INFO = plsc.get_sparse_core_info()
NC = _INFO.num_cores
NS = _INFO.num_subcores
NW = NC * NS
BW = B // NW
CH = 16
NCH = BW // CH
CTX_PER_CH = CH * 2 * W
WN_PER_CH = CH * (1 + NEG)


def _sig_clamped(x):
    s = 1.0 / (1.0 + jnp.exp(-x))
    s = jnp.where(x > 6.0, 1.0, s)
    return jnp.where(x > -6.0, s, 0.0)


def _cbow_body(emb0_hbm, emb1_hbm, ctx_idx_hbm, wn_idx_hbm, lens_hbm, mask_hbm,
               out_hbm,
               ctx_idx_v, wn_idx_v, lens_v, mask_v, ctxbuf, wnbuf, lossbuf,
               sem):
    wid = lax.axis_index("s") * NC + lax.axis_index("c")

    pltpu.sync_copy(ctx_idx_hbm.at[wid], ctx_idx_v)
    pltpu.sync_copy(wn_idx_hbm.at[wid], wn_idx_v)
    pltpu.sync_copy(lens_hbm.at[wid], lens_v)
    pltpu.sync_copy(mask_hbm.at[wid], mask_v)

    iota = lax.iota(jnp.int32, 16)
    ctx_rows = [iota * (2 * W) + k for k in range(2 * W)]
    wn_rows = [iota * (1 + NEG) + r for r in range(1 + NEG)]

    def chunk(k, loss_acc):
        pltpu.async_copy(emb0_hbm.at[ctx_idx_v.at[2 * k]],
                         ctxbuf.at[pl.ds(0, 80)], sem)
        pltpu.async_copy(emb0_hbm.at[ctx_idx_v.at[2 * k + 1]],
                         ctxbuf.at[pl.ds(80, 80)], sem)
        pltpu.async_copy(emb1_hbm.at[wn_idx_v.at[k]], wnbuf, sem).wait()
        pltpu.make_async_copy(emb0_hbm.at[pl.ds(0, CTX_PER_CH)], ctxbuf,
                              sem).wait()

        def dbody(d, accs):
            col = jnp.full((16,), d, jnp.int32)
            csum = plsc.load_gather(ctxbuf, [ctx_rows[0], col])
            for kk in range(1, 2 * W):
                csum = csum + plsc.load_gather(ctxbuf, [ctx_rows[kk], col])
            out = []
            for r in range(1 + NEG):
                v = plsc.load_gather(wnbuf, [wn_rows[r], col])
                out.append(accs[r] + csum * v)
            return tuple(out)

        zero = jnp.zeros((16,), jnp.float32)
        accs = lax.fori_loop(0, D, dbody, (zero,) * (1 + NEG))

        inv_len = 1.0 / lens_v[pl.ds(k * CH, CH)]
        pos = _sig_clamped(accs[0] * inv_len)
        loss = loss_acc + 0.5 * (1.0 - pos) * (1.0 - pos)
        for r in range(NEG):
            neg = _sig_clamped(accs[1 + r] * inv_len)
            neg = neg * mask_v[r, pl.ds(k * CH, CH)]
            loss = loss + 0.5 * neg * neg
        return loss

    loss = lax.fori_loop(0, NCH, chunk, jnp.zeros((16,), jnp.float32))
    lossbuf[...] = loss
    pltpu.sync_copy(lossbuf, out_hbm.at[wid])


@jax.jit
def kernel(emb0_weight, emb1_weight, data):
    d32 = data.astype(jnp.int32)
    ctx = d32[:, : 2 * W]
    lens = d32[:, 2 * W].astype(jnp.float32)
    wn = d32[:, 2 * W + 1 : 2 * W + 2 + NEG]
    mask = d32[:, 2 * W + 2 + NEG :].astype(jnp.float32)

    ctx_idx = ctx.reshape(NW, NCH * 2, CTX_PER_CH // 2)
    wn_idx = wn.reshape(NW, NCH, WN_PER_CH)
    lens_r = lens.reshape(NW, BW)
    mask_r = mask.T.reshape(NEG, NW, BW).transpose(1, 0, 2)

    run = pl.kernel(
        _cbow_body,
        out_type=jax.ShapeDtypeStruct((NW, 16), jnp.float32),
        mesh=plsc.VectorSubcoreMesh(core_axis_name="c", subcore_axis_name="s"),
        compiler_params=pltpu.CompilerParams(
            needs_layout_passes=False, use_tc_tiling_on_sc=False
        ),
        scratch_types=[
            pltpu.VMEM((NCH * 2, CTX_PER_CH // 2), jnp.int32),
            pltpu.VMEM((NCH, WN_PER_CH), jnp.int32),
            pltpu.VMEM((BW,), jnp.float32),
            pltpu.VMEM((NEG, BW), jnp.float32),
            pltpu.VMEM((CTX_PER_CH, D), jnp.float32),
            pltpu.VMEM((WN_PER_CH, D), jnp.float32),
            pltpu.VMEM((16,), jnp.float32),
            pltpu.SemaphoreType.DMA,
        ],
    )
    partials = run(emb0_weight, emb1_weight, ctx_idx, wn_idx, lens_r, mask_r)
    return jnp.sum(partials)


# speedup vs baseline: 1.8378x; 1.1985x over previous
import functools

import jax
import jax.numpy as jnp
from jax import lax
from jax.experimental import pallas as pl
from jax.experimental.pallas import tpu as pltpu
from jax.experimental.pallas import tpu_sc as plsc

V = 1000000
D = 64
W = 5
NEG = 5
B = 16384
NCTX = 2 * W
NWN = 1 + NEG

_INFO = plsc.get_sparse_core_info()
NC = _INFO.num_cores
NS = _INFO.num_subcores
NW = NC * NS
BW = B // NW
CH = 16
NCH = BW // CH
CTX_PER_CH = CH * NCTX
WN_PER_CH = CH * NWN


def _sig_clamped(x):
    s = 1.0 / (1.0 + jnp.exp(-x))
    s = jnp.where(x > 6.0, 1.0, s)
    return jnp.where(x > -6.0, s, 0.0)


def _cbow_body(emb0_hbm, emb1_hbm, ctx_t_hbm, wn_t_hbm, lens_hbm, mask_hbm,
               out_hbm,
               ctxidx_v, wnidx_v, physctx, physwn, lens_v, mask_v,
               ctxbuf, wnbuf, lossbuf, sem):
    wid = lax.axis_index("s") * NC + lax.axis_index("c")

    pltpu.sync_copy(ctx_t_hbm.at[wid], ctxidx_v)
    pltpu.sync_copy(wn_t_hbm.at[wid], wnidx_v)
    pltpu.sync_copy(lens_hbm.at[wid], lens_v)
    pltpu.sync_copy(mask_hbm.at[wid], mask_v)

    iota = lax.iota(jnp.int32, 16)

    def prep(c, carry):
        for k in range(NCTX):
            idx = ctxidx_v[k, pl.ds(c * CH, CH)]
            phys = jnp.where(idx == V, 0, idx >> 1)
            physctx[2 * c + k // 5, pl.ds((k % 5) * CH, CH)] = phys
        for r in range(NWN):
            idx = wnidx_v[r, pl.ds(c * CH, CH)]
            physwn[c, pl.ds(r * CH, CH)] = idx >> 1
        return carry

    lax.fori_loop(0, NCH, prep, 0)

    rows_c = [iota + CH * k for k in range(NCTX)]
    rows_w = [iota + CH * r for r in range(NWN)]

    def chunk(c, loss_acc):
        pltpu.async_copy(emb0_hbm.at[physctx.at[2 * c]],
                         ctxbuf.at[pl.ds(0, 80)], sem)
        pltpu.async_copy(emb0_hbm.at[physctx.at[2 * c + 1]],
                         ctxbuf.at[pl.ds(80, 80)], sem)
        pltpu.async_copy(emb1_hbm.at[physwn.at[c]], wnbuf, sem).wait()
        pltpu.make_async_copy(emb0_hbm.at[pl.ds(0, CTX_PER_CH)], ctxbuf,
                              sem).wait()

        offc, valc, offw = [], [], []
        for k in range(NCTX):
            idx = ctxidx_v[k, pl.ds(c * CH, CH)]
            offc.append((idx & 1) * D)
            valc.append(jnp.where(idx == V, 0.0, 1.0))
        for r in range(NWN):
            idx = wnidx_v[r, pl.ds(c * CH, CH)]
            offw.append((idx & 1) * D)

        def dbody(d, accs):
            col = (d + iota) & (D - 1)
            csum = plsc.load_gather(ctxbuf, [rows_c[0], col + offc[0]]) * valc[0]
            for k in range(1, NCTX):
                v = plsc.load_gather(ctxbuf, [rows_c[k], col + offc[k]])
                csum = csum + v * valc[k]
            out = []
            for r in range(NWN):
                v = plsc.load_gather(wnbuf, [rows_w[r], col + offw[r]])
                out.append(accs[r] + csum * v)
            return tuple(out)

        zero = jnp.zeros((16,), jnp.float32)
        accs = lax.fori_loop(0, D, dbody, (zero,) * NWN)

        inv_len = 1.0 / lens_v[pl.ds(c * CH, CH)]
        pos = _sig_clamped(accs[0] * inv_len)
        loss = loss_acc + 0.5 * (1.0 - pos) * (1.0 - pos)
        for r in range(NEG):
            neg = _sig_clamped(accs[1 + r] * inv_len)
            neg = neg * mask_v[r, pl.ds(c * CH, CH)]
            loss = loss + 0.5 * neg * neg
        return loss

    loss = lax.fori_loop(0, NCH, chunk, jnp.zeros((16,), jnp.float32))
    lossbuf[...] = loss
    pltpu.sync_copy(lossbuf, out_hbm.at[wid])


@jax.jit
def kernel(emb0_weight, emb1_weight, data):
    d32 = data.astype(jnp.int32)
    ctx = d32[:, :NCTX]
    lens = d32[:, NCTX].astype(jnp.float32)
    wn = d32[:, NCTX + 1 : NCTX + 1 + NWN]
    mask = d32[:, NCTX + 1 + NWN :].astype(jnp.float32)

    emb0_r = emb0_weight[:V].reshape(V // 2, 2 * D)
    emb1_r = emb1_weight.reshape(V // 2, 2 * D)

    ctx_t = ctx.T.reshape(NCTX, NW, BW).transpose(1, 0, 2)
    wn_t = wn.T.reshape(NWN, NW, BW).transpose(1, 0, 2)
    lens_r = lens.reshape(NW, BW)
    mask_r = mask.T.reshape(NEG, NW, BW).transpose(1, 0, 2)

    run = pl.kernel(
        _cbow_body,
        out_type=jax.ShapeDtypeStruct((NW, 16), jnp.float32),
        mesh=plsc.VectorSubcoreMesh(core_axis_name="c", subcore_axis_name="s"),
        compiler_params=pltpu.CompilerParams(
            needs_layout_passes=False, use_tc_tiling_on_sc=True
        ),
        scratch_types=[
            pltpu.VMEM((NCTX, BW), jnp.int32),
            pltpu.VMEM((NWN, BW), jnp.int32),
            pltpu.VMEM((NCH * 2, CTX_PER_CH // 2), jnp.int32),
            pltpu.VMEM((NCH, WN_PER_CH), jnp.int32),
            pltpu.VMEM((BW,), jnp.float32),
            pltpu.VMEM((NEG, BW), jnp.float32),
            pltpu.VMEM((CTX_PER_CH, 2 * D), jnp.float32),
            pltpu.VMEM((WN_PER_CH, 2 * D), jnp.float32),
            pltpu.VMEM((16,), jnp.float32),
            pltpu.SemaphoreType.DMA,
        ],
    )
    partials = run(emb0_r, emb1_r, ctx_t, wn_t, lens_r, mask_r)
    return jnp.sum(partials)
